# baseline (speedup 1.0000x reference)
import jax
import jax.numpy as jnp
from jax import lax
from jax.experimental import pallas as pl
from jax.experimental.pallas import tpu as pltpu
from jax.experimental.pallas import tpu_sc as plsc

L = 16
NC = 2
NS = 16
NW = NC * NS
N_COMP = 5
C_DIM = 64
NSLOT = 2


def _tce_body(x_hbm, comp_hbm, e0, e1, e2, e3, e4,
              o0, o1, o2, o3, o4,
              x_v, cidx_v, cvals_v, rows_v, semc, semg, semw):
    embs = (e0, e1, e2, e3, e4)
    outs = (o0, o1, o2, o3, o4)
    batch = x_hbm.shape[0]
    t_vocab = comp_hbm.shape[0] // N_COMP
    per_w = batch // NW

    wid = lax.axis_index("s") * NC + lax.axis_index("c")
    base = wid * per_w

    pltpu.sync_copy(x_hbm.at[pl.ds(base, per_w)], x_v)
    for i in range(N_COMP):
        off = jnp.int32(i * t_vocab)
        for j in range(per_w // L):
            cidx_v[pl.ds(i * per_w + j * L, L)] = x_v[pl.ds(j * L, L)] + off

    comp_dmas = [
        pltpu.async_copy(
            comp_hbm.at[cidx_v.at[pl.ds(i * per_w, per_w)]],
            cvals_v.at[pl.ds(i * per_w, per_w)], semc)
        for i in range(N_COMP)
    ]
    for d in comp_dmas:
        d.wait()

    gd = [None] * N_COMP
    wd = [None] * N_COMP

    def fire(i):
        gd[i] = pltpu.async_copy(
            embs[i].at[cvals_v.at[pl.ds(i * per_w, per_w)]],
            rows_v.at[i % NSLOT], semg.at[i % NSLOT])

    for i in range(min(NSLOT, N_COMP)):
        fire(i)
    for i in range(N_COMP):
        gd[i].wait()
        wd[i] = pltpu.async_copy(
            rows_v.at[i % NSLOT], outs[i].at[pl.ds(base, per_w)],
            semw.at[i % NSLOT])
        if i + NSLOT < N_COMP:
            wd[i].wait()
            fire(i + NSLOT)
    for i in range(max(0, N_COMP - NSLOT), N_COMP):
        wd[i].wait()


def kernel(x, comp_table, emb0, emb1, emb2, emb3, emb4):
    batch = x.shape[0]
    per_w = batch // NW
    comp_cm = comp_table.T.reshape(-1)
    embs = tuple(e.at[0].set(0.0) for e in (emb0, emb1, emb2, emb3, emb4))

    mesh = plsc.VectorSubcoreMesh(core_axis_name="c", subcore_axis_name="s")
    out_type = tuple(
        jax.ShapeDtypeStruct((batch, C_DIM), jnp.float32) for _ in range(N_COMP)
    )
    scratch = [
        pltpu.VMEM((per_w,), jnp.int32),
        pltpu.VMEM((N_COMP * per_w,), jnp.int32),
        pltpu.VMEM((N_COMP * per_w,), jnp.int32),
        pltpu.VMEM((NSLOT, per_w, C_DIM), jnp.float32),
        pltpu.SemaphoreType.DMA,
        pltpu.SemaphoreType.DMA((NSLOT,)),
        pltpu.SemaphoreType.DMA((NSLOT,)),
    ]
    f = pl.kernel(
        _tce_body, mesh=mesh, out_type=out_type, scratch_types=scratch,
        compiler_params=pltpu.CompilerParams(use_tc_tiling_on_sc=False),
    )
    return f(x, comp_cm, *embs)


# speedup vs baseline: 2.6411x; 2.6411x over previous
import jax
import jax.numpy as jnp
from jax import lax
from jax.experimental import pallas as pl
from jax.experimental.pallas import tpu as pltpu
from jax.experimental.pallas import tpu_sc as plsc

L = 16
NC = 2
NS = 16
NW = NC * NS
N_COMP = 5
C_DIM = 64
NSLOT = 2


def _tce_body(x_hbm, comp_hbm, e0, e1, e2, e3, e4,
              o0, o1, o2, o3, o4,
              x_v, cidx_v, cvals_v, rows_v, semc, semg, semw):
    embs = (e0, e1, e2, e3, e4)
    outs = (o0, o1, o2, o3, o4)
    batch = x_hbm.shape[0]
    t_vocab = comp_hbm.shape[0] // N_COMP
    per_w = batch // NW

    wid = lax.axis_index("s") * NC + lax.axis_index("c")
    base = wid * per_w

    pltpu.sync_copy(x_hbm.at[pl.ds(base, per_w)], x_v)
    for i in range(N_COMP):
        off = jnp.int32(i * t_vocab)
        for j in range(per_w // L):
            cidx_v[pl.ds(i * per_w + j * L, L)] = x_v[pl.ds(j * L, L)] + off

    comp_dmas = [
        pltpu.async_copy(
            comp_hbm.at[cidx_v.at[pl.ds(i * per_w, per_w)]],
            cvals_v.at[pl.ds(i * per_w, per_w)], semc)
        for i in range(N_COMP)
    ]
    for d in comp_dmas:
        d.wait()

    gd = [None] * N_COMP
    wd = [None] * N_COMP

    def fire(i):
        gd[i] = pltpu.async_copy(
            embs[i].at[cvals_v.at[pl.ds(i * per_w, per_w)]],
            rows_v.at[i % NSLOT], semg.at[i % NSLOT])

    for i in range(min(NSLOT, N_COMP)):
        fire(i)
    for i in range(N_COMP):
        gd[i].wait()
        wd[i] = pltpu.async_copy(
            rows_v.at[i % NSLOT], outs[i].at[pl.ds(base, per_w)],
            semw.at[i % NSLOT])
        if i + NSLOT < N_COMP:
            wd[i].wait()
            fire(i + NSLOT)
    for i in range(max(0, N_COMP - NSLOT), N_COMP):
        wd[i].wait()


def kernel(x, comp_table, emb0, emb1, emb2, emb3, emb4):
    batch = x.shape[0]
    per_w = batch // NW
    t_vocab = comp_table.shape[0]
    reps = [max(1, min(512, 2048 // e.shape[0])) for e in
            (emb0, emb1, emb2, emb3, emb4)]
    embs = tuple(
        jnp.tile(e.at[0].set(0.0), (r, 1))
        for e, r in zip((emb0, emb1, emb2, emb3, emb4), reps)
    )
    t_ids = jnp.arange(t_vocab, dtype=jnp.int32)
    cols = [
        comp_table[:, i] + (t_ids % reps[i]) * e.shape[0]
        for i, e in enumerate((emb0, emb1, emb2, emb3, emb4))
    ]
    comp_cm = jnp.concatenate(cols)

    mesh = plsc.VectorSubcoreMesh(core_axis_name="c", subcore_axis_name="s")
    out_type = tuple(
        jax.ShapeDtypeStruct((batch, C_DIM), jnp.float32) for _ in range(N_COMP)
    )
    scratch = [
        pltpu.VMEM((per_w,), jnp.int32),
        pltpu.VMEM((N_COMP * per_w,), jnp.int32),
        pltpu.VMEM((N_COMP * per_w,), jnp.int32),
        pltpu.VMEM((NSLOT, per_w, C_DIM), jnp.float32),
        pltpu.SemaphoreType.DMA,
        pltpu.SemaphoreType.DMA((NSLOT,)),
        pltpu.SemaphoreType.DMA((NSLOT,)),
    ]
    f = pl.kernel(
        _tce_body, mesh=mesh, out_type=out_type, scratch_types=scratch,
        compiler_params=pltpu.CompilerParams(use_tc_tiling_on_sc=False),
    )
    return f(x, comp_cm, *embs)
